# baseline (speedup 1.0000x reference)
import functools
import jax
import jax.numpy as jnp
from jax import lax
from jax.experimental import pallas as pl
from jax.experimental.pallas import tpu as pltpu
from jax.experimental.pallas import tpu_sc as plsc

_D = 256
_HW = 4 * _D
_W = 2 * _HW
_N_DRUG = 1373
_B = 16384
_NC = 2
_NS = 16
_NW = _NC * _NS
_CB = 16
_BPW = _B // _NW
_NCHUNK = _BPW // _CB
_L = 16


def _body(tab, x1, x2, out, x1_v, x2_v, bigA, bigB, gsA, gsB, wsA, wsB):
    wid = lax.axis_index("s") * _NC + lax.axis_index("c")
    base_w = wid * _BPW
    bufs = (bigA, bigB)
    gsems = (gsA, gsB)
    wsems = (wsA, wsB)

    pltpu.sync_copy(x1.at[pl.ds(base_w, _BPW)], x1_v)
    pltpu.sync_copy(x2.at[pl.ds(base_w, _BPW)], x2_v)
    for j in range(_BPW // _L):
        x2_v[pl.ds(j * _L, _L)] = x2_v[pl.ds(j * _L, _L)] + _N_DRUG

    def fire_gathers(c, buf, sem):
        off = c * _CB
        pltpu.async_copy(tab.at[x1_v.at[pl.ds(off, _CB)]],
                         buf.at[:, pl.ds(0, _HW)], sem)
        pltpu.async_copy(tab.at[x2_v.at[pl.ds(off, _CB)]],
                         buf.at[:, pl.ds(_HW, _HW)], sem)

    def drain_gathers(buf, sem):
        pltpu.make_async_copy(out.at[pl.ds(0, _CB), :], buf, sem).wait()

    def fire_write(c, buf, sem):
        pltpu.async_copy(buf, out.at[pl.ds(base_w + c * _CB, _CB), :], sem)

    def drain_write(buf, sem):
        pltpu.make_async_copy(out.at[pl.ds(0, _CB), :], buf, sem).wait()

    fire_gathers(0, bufs[0], gsems[0])
    drain_gathers(bufs[0], gsems[0])
    fire_write(0, bufs[0], wsems[0])
    fire_gathers(1, bufs[1], gsems[1])

    def outer(o, _):
        for step in range(2):
            c = 2 * o + 1 + step
            x = (1 + step) % 2
            y = 1 - x
            drain_gathers(bufs[x], gsems[x])
            fire_write(c, bufs[x], wsems[x])
            drain_write(bufs[y], wsems[y])
            fire_gathers(c + 1, bufs[y], gsems[y])
        return ()

    lax.fori_loop(0, (_NCHUNK - 2) // 2, outer, (), unroll=False)

    cl = _NCHUNK - 1
    xl = cl % 2
    yl = 1 - xl
    drain_gathers(bufs[xl], gsems[xl])
    fire_write(cl, bufs[xl], wsems[xl])
    drain_write(bufs[yl], wsems[yl])
    drain_write(bufs[xl], wsems[xl])


@jax.jit
def _run(h1, h2, d1, d2, m1, m2, x1, x2):
    tab = jnp.concatenate([
        jnp.concatenate([d1, d2, h1[:_N_DRUG], h2[:_N_DRUG]], axis=1),
        jnp.concatenate([m1, m2, h1[_N_DRUG:], h2[_N_DRUG:]], axis=1),
    ], axis=0)
    mesh = plsc.VectorSubcoreMesh(core_axis_name="c", subcore_axis_name="s")
    f = pl.kernel(
        _body,
        out_type=jax.ShapeDtypeStruct((_B, _W), jnp.float32),
        mesh=mesh,
        scratch_types=[
            pltpu.VMEM((_BPW,), jnp.int32),
            pltpu.VMEM((_BPW,), jnp.int32),
            pltpu.VMEM((_CB, _W), jnp.float32),
            pltpu.VMEM((_CB, _W), jnp.float32),
            pltpu.SemaphoreType.DMA,
            pltpu.SemaphoreType.DMA,
            pltpu.SemaphoreType.DMA,
            pltpu.SemaphoreType.DMA,
        ],
    )
    return f(tab, x1, x2)


def kernel(hete_1hop, hete_2hop, drug_homo_1hop, drug_homo_2hop,
           mic_homo_1hop, mic_homo_2hop, x1, x2):
    out = _run(hete_1hop, hete_2hop, drug_homo_1hop, drug_homo_2hop,
               mic_homo_1hop, mic_homo_2hop,
               x1.astype(jnp.int32), x2.astype(jnp.int32))
    return out.reshape(_B, 1, 2, _HW)


# speedup vs baseline: 2.1717x; 2.1717x over previous
import functools
import jax
import jax.numpy as jnp
from jax import lax
from jax.experimental import pallas as pl
from jax.experimental.pallas import tpu as pltpu
from jax.experimental.pallas import tpu_sc as plsc

_D = 256
_HW = 4 * _D
_W = 2 * _HW
_N_DRUG = 1373
_B = 16384
_NC = 2
_NS = 16
_NW = _NC * _NS
_CB = 16
_BPW = _B // _NW
_NCHUNK = _BPW // _CB
_L = 16


def _body(tab, x1, x2, out, x1_v, x2_v, dA, mA, dB, mB,
          gsA, gsB, wsA, wsB):
    wid = lax.axis_index("s") * _NC + lax.axis_index("c")
    base_w = wid * _BPW
    bufs = ((dA, mA), (dB, mB))
    gsems = (gsA, gsB)
    wsems = (wsA, wsB)

    pltpu.sync_copy(x1.at[pl.ds(base_w, _BPW)], x1_v)
    pltpu.sync_copy(x2.at[pl.ds(base_w, _BPW)], x2_v)
    for j in range(_BPW // _L):
        x2_v[pl.ds(j * _L, _L)] = x2_v[pl.ds(j * _L, _L)] + _N_DRUG

    def fire_gathers(c, buf, sem):
        off = c * _CB
        pltpu.async_copy(tab.at[x1_v.at[pl.ds(off, _CB)]], buf[0], sem)
        pltpu.async_copy(tab.at[x2_v.at[pl.ds(off, _CB)]], buf[1], sem)

    def drain_gathers(buf, sem):
        pltpu.make_async_copy(tab.at[pl.ds(0, _CB)], buf[0], sem).wait()
        pltpu.make_async_copy(tab.at[pl.ds(0, _CB)], buf[1], sem).wait()

    def fire_write(c, buf, sem):
        b = base_w + c * _CB
        pltpu.async_copy(buf[0], out.at[pl.ds(b, _CB), 0, 0], sem)
        pltpu.async_copy(buf[1], out.at[pl.ds(b, _CB), 0, 1], sem)

    def drain_write(buf, sem):
        pltpu.make_async_copy(tab.at[pl.ds(0, _CB)], buf[0], sem).wait()
        pltpu.make_async_copy(tab.at[pl.ds(0, _CB)], buf[1], sem).wait()

    fire_gathers(0, bufs[0], gsems[0])
    drain_gathers(bufs[0], gsems[0])
    fire_write(0, bufs[0], wsems[0])
    fire_gathers(1, bufs[1], gsems[1])

    def outer(o, _):
        for step in range(2):
            c = 2 * o + 1 + step
            x = (1 + step) % 2
            y = 1 - x
            drain_gathers(bufs[x], gsems[x])
            fire_write(c, bufs[x], wsems[x])
            drain_write(bufs[y], wsems[y])
            fire_gathers(c + 1, bufs[y], gsems[y])
        return ()

    lax.fori_loop(0, (_NCHUNK - 2) // 2, outer, (), unroll=False)

    cl = _NCHUNK - 1
    xl = cl % 2
    yl = 1 - xl
    drain_gathers(bufs[xl], gsems[xl])
    fire_write(cl, bufs[xl], wsems[xl])
    drain_write(bufs[yl], wsems[yl])
    drain_write(bufs[xl], wsems[xl])


@jax.jit
def _run(h1, h2, d1, d2, m1, m2, x1, x2):
    tab = jnp.concatenate([
        jnp.concatenate([d1, d2, h1[:_N_DRUG], h2[:_N_DRUG]], axis=1),
        jnp.concatenate([m1, m2, h1[_N_DRUG:], h2[_N_DRUG:]], axis=1),
    ], axis=0)
    mesh = plsc.VectorSubcoreMesh(core_axis_name="c", subcore_axis_name="s")
    f = pl.kernel(
        _body,
        out_type=jax.ShapeDtypeStruct((_B, 1, 2, _HW), jnp.float32),
        mesh=mesh,
        scratch_types=[
            pltpu.VMEM((_BPW,), jnp.int32),
            pltpu.VMEM((_BPW,), jnp.int32),
            pltpu.VMEM((_CB, _HW), jnp.float32),
            pltpu.VMEM((_CB, _HW), jnp.float32),
            pltpu.VMEM((_CB, _HW), jnp.float32),
            pltpu.VMEM((_CB, _HW), jnp.float32),
            pltpu.SemaphoreType.DMA,
            pltpu.SemaphoreType.DMA,
            pltpu.SemaphoreType.DMA,
            pltpu.SemaphoreType.DMA,
        ],
    )
    return f(tab, x1, x2)


def kernel(hete_1hop, hete_2hop, drug_homo_1hop, drug_homo_2hop,
           mic_homo_1hop, mic_homo_2hop, x1, x2):
    return _run(hete_1hop, hete_2hop, drug_homo_1hop, drug_homo_2hop,
                mic_homo_1hop, mic_homo_2hop,
                x1.astype(jnp.int32), x2.astype(jnp.int32))


# speedup vs baseline: 2.3529x; 1.0834x over previous
import functools
import jax
import jax.numpy as jnp
from jax import lax
from jax.experimental import pallas as pl
from jax.experimental.pallas import tpu as pltpu
from jax.experimental.pallas import tpu_sc as plsc

_D = 256
_HW = 4 * _D
_W = 2 * _HW
_N_DRUG = 1373
_B = 16384
_NC = 2
_NS = 16
_NW = _NC * _NS
_CB = 16
_BPW = _B // _NW
_NCHUNK = _BPW // _CB
_L = 16


def _body(h1, h2, d1, d2, m1, m2, x1, x2, out, x1_v, x2_v, x2h_v,
          dA, mA, dB, mB, gsA, gsB, wsA, wsB):
    wid = lax.axis_index("s") * _NC + lax.axis_index("c")
    base_w = wid * _BPW
    bufs = ((dA, mA), (dB, mB))
    gsems = (gsA, gsB)
    wsems = (wsA, wsB)

    pltpu.sync_copy(x1.at[pl.ds(base_w, _BPW)], x1_v)
    pltpu.sync_copy(x2.at[pl.ds(base_w, _BPW)], x2_v)
    for j in range(_BPW // _L):
        x2h_v[pl.ds(j * _L, _L)] = x2_v[pl.ds(j * _L, _L)] + _N_DRUG

    def fire_gathers(c, buf, sem):
        off = c * _CB
        i1 = x1_v.at[pl.ds(off, _CB)]
        i2 = x2_v.at[pl.ds(off, _CB)]
        i2h = x2h_v.at[pl.ds(off, _CB)]
        for k, tab, idx in ((0, d1, i1), (1, d2, i1), (2, h1, i1),
                            (3, h2, i1), (0, m1, i2), (1, m2, i2),
                            (2, h1, i2h), (3, h2, i2h)):
            half = buf[0] if idx is i1 else buf[1]
            pltpu.async_copy(tab.at[idx], half.at[:, pl.ds(k * _D, _D)],
                             sem)

    def drain_gathers(buf, sem):
        dummy = out.at[pl.ds(0, _CB), 0, 0]
        pltpu.make_async_copy(dummy, buf[0], sem).wait()
        pltpu.make_async_copy(dummy, buf[1], sem).wait()

    def fire_write(c, buf, sem):
        b = base_w + c * _CB
        pltpu.async_copy(buf[0], out.at[pl.ds(b, _CB), 0, 0], sem)
        pltpu.async_copy(buf[1], out.at[pl.ds(b, _CB), 0, 1], sem)

    def drain_write(buf, sem):
        dummy = out.at[pl.ds(0, _CB), 0, 0]
        pltpu.make_async_copy(dummy, buf[0], sem).wait()
        pltpu.make_async_copy(dummy, buf[1], sem).wait()

    fire_gathers(0, bufs[0], gsems[0])
    drain_gathers(bufs[0], gsems[0])
    fire_write(0, bufs[0], wsems[0])
    fire_gathers(1, bufs[1], gsems[1])

    def outer(o, _):
        for step in range(2):
            c = 2 * o + 1 + step
            x = (1 + step) % 2
            y = 1 - x
            drain_gathers(bufs[x], gsems[x])
            fire_write(c, bufs[x], wsems[x])
            drain_write(bufs[y], wsems[y])
            fire_gathers(c + 1, bufs[y], gsems[y])
        return ()

    lax.fori_loop(0, (_NCHUNK - 2) // 2, outer, (), unroll=False)

    cl = _NCHUNK - 1
    xl = cl % 2
    yl = 1 - xl
    drain_gathers(bufs[xl], gsems[xl])
    fire_write(cl, bufs[xl], wsems[xl])
    drain_write(bufs[yl], wsems[yl])
    drain_write(bufs[xl], wsems[xl])


@jax.jit
def _run(h1, h2, d1, d2, m1, m2, x1, x2):
    mesh = plsc.VectorSubcoreMesh(core_axis_name="c", subcore_axis_name="s")
    f = pl.kernel(
        _body,
        out_type=jax.ShapeDtypeStruct((_B, 1, 2, _HW), jnp.float32),
        mesh=mesh,
        scratch_types=[
            pltpu.VMEM((_BPW,), jnp.int32),
            pltpu.VMEM((_BPW,), jnp.int32),
            pltpu.VMEM((_BPW,), jnp.int32),
            pltpu.VMEM((_CB, _HW), jnp.float32),
            pltpu.VMEM((_CB, _HW), jnp.float32),
            pltpu.VMEM((_CB, _HW), jnp.float32),
            pltpu.VMEM((_CB, _HW), jnp.float32),
            pltpu.SemaphoreType.DMA,
            pltpu.SemaphoreType.DMA,
            pltpu.SemaphoreType.DMA,
            pltpu.SemaphoreType.DMA,
        ],
    )
    return f(h1, h2, d1, d2, m1, m2, x1, x2)


def kernel(hete_1hop, hete_2hop, drug_homo_1hop, drug_homo_2hop,
           mic_homo_1hop, mic_homo_2hop, x1, x2):
    return _run(hete_1hop, hete_2hop, drug_homo_1hop, drug_homo_2hop,
                mic_homo_1hop, mic_homo_2hop,
                x1.astype(jnp.int32), x2.astype(jnp.int32))


# speedup vs baseline: 2.4423x; 1.0380x over previous
import functools
import jax
import jax.numpy as jnp
from jax import lax
from jax.experimental import pallas as pl
from jax.experimental.pallas import tpu as pltpu
from jax.experimental.pallas import tpu_sc as plsc

_D = 256
_HW = 4 * _D
_W = 2 * _HW
_N_DRUG = 1373
_B = 16384
_NC = 2
_NS = 16
_NW = _NC * _NS
_CB = 16
_BPW = _B // _NW
_NCHUNK = _BPW // _CB
_L = 16


def _body(h1, h2, d1, d2, m1, m2, x1, x2, out, x1_v, x2_v, x2h_v,
          dA, mA, dB, mB, dC, mC, gsA, gsB, gsC, wsA, wsB, wsC):
    wid = lax.axis_index("s") * _NC + lax.axis_index("c")
    base_w = wid * _BPW
    bufs = ((dA, mA), (dB, mB), (dC, mC))
    gsems = (gsA, gsB, gsC)
    wsems = (wsA, wsB, wsC)

    pltpu.sync_copy(x1.at[pl.ds(base_w, _BPW)], x1_v)
    pltpu.sync_copy(x2.at[pl.ds(base_w, _BPW)], x2_v)
    for j in range(_BPW // _L):
        x2h_v[pl.ds(j * _L, _L)] = x2_v[pl.ds(j * _L, _L)] + _N_DRUG

    def fire_gathers(c, buf, sem):
        off = c * _CB
        i1 = x1_v.at[pl.ds(off, _CB)]
        i2 = x2_v.at[pl.ds(off, _CB)]
        i2h = x2h_v.at[pl.ds(off, _CB)]
        for k, tab, idx in ((0, d1, i1), (1, d2, i1), (2, h1, i1),
                            (3, h2, i1), (0, m1, i2), (1, m2, i2),
                            (2, h1, i2h), (3, h2, i2h)):
            half = buf[0] if idx is i1 else buf[1]
            pltpu.async_copy(tab.at[idx], half.at[:, pl.ds(k * _D, _D)],
                             sem)

    def drain_gathers(buf, sem):
        dummy = out.at[pl.ds(0, _CB), 0, 0]
        pltpu.make_async_copy(dummy, buf[0], sem).wait()
        pltpu.make_async_copy(dummy, buf[1], sem).wait()

    def fire_write(c, buf, sem):
        b = base_w + c * _CB
        pltpu.async_copy(buf[0], out.at[pl.ds(b, _CB), 0, 0], sem)
        pltpu.async_copy(buf[1], out.at[pl.ds(b, _CB), 0, 1], sem)

    def drain_write(buf, sem):
        dummy = out.at[pl.ds(0, _CB), 0, 0]
        pltpu.make_async_copy(dummy, buf[0], sem).wait()
        pltpu.make_async_copy(dummy, buf[1], sem).wait()

    fire_gathers(0, bufs[0], gsems[0])
    fire_gathers(1, bufs[1], gsems[1])
    drain_gathers(bufs[0], gsems[0])
    fire_write(0, bufs[0], wsems[0])
    fire_gathers(2, bufs[2], gsems[2])
    drain_gathers(bufs[1], gsems[1])
    fire_write(1, bufs[1], wsems[1])
    drain_write(bufs[0], wsems[0])
    fire_gathers(3, bufs[0], gsems[0])

    def outer(o, _):
        for step in range(3):
            c = 2 + 3 * o + step
            x = (2 + step) % 3
            y = (x + 2) % 3
            drain_gathers(bufs[x], gsems[x])
            fire_write(c, bufs[x], wsems[x])
            drain_write(bufs[y], wsems[y])
            fire_gathers(c + 2, bufs[y], gsems[y])
        return ()

    lax.fori_loop(0, (_NCHUNK - 5) // 3, outer, (), unroll=False)

    c = _NCHUNK - 3
    x = c % 3
    y = (x + 2) % 3
    drain_gathers(bufs[x], gsems[x])
    fire_write(c, bufs[x], wsems[x])
    drain_write(bufs[y], wsems[y])
    fire_gathers(_NCHUNK - 1, bufs[y], gsems[y])
    for c in (_NCHUNK - 2, _NCHUNK - 1):
        x = c % 3
        drain_gathers(bufs[x], gsems[x])
        fire_write(c, bufs[x], wsems[x])
    for x in range(3):
        drain_write(bufs[x], wsems[x])


@jax.jit
def _run(h1, h2, d1, d2, m1, m2, x1, x2):
    mesh = plsc.VectorSubcoreMesh(core_axis_name="c", subcore_axis_name="s")
    f = pl.kernel(
        _body,
        out_type=jax.ShapeDtypeStruct((_B, 1, 2, _HW), jnp.float32),
        mesh=mesh,
        scratch_types=[
            pltpu.VMEM((_BPW,), jnp.int32),
            pltpu.VMEM((_BPW,), jnp.int32),
            pltpu.VMEM((_BPW,), jnp.int32),
            pltpu.VMEM((_CB, _HW), jnp.float32),
            pltpu.VMEM((_CB, _HW), jnp.float32),
            pltpu.VMEM((_CB, _HW), jnp.float32),
            pltpu.VMEM((_CB, _HW), jnp.float32),
            pltpu.VMEM((_CB, _HW), jnp.float32),
            pltpu.VMEM((_CB, _HW), jnp.float32),
            pltpu.SemaphoreType.DMA,
            pltpu.SemaphoreType.DMA,
            pltpu.SemaphoreType.DMA,
            pltpu.SemaphoreType.DMA,
            pltpu.SemaphoreType.DMA,
            pltpu.SemaphoreType.DMA,
        ],
    )
    return f(h1, h2, d1, d2, m1, m2, x1, x2)


def kernel(hete_1hop, hete_2hop, drug_homo_1hop, drug_homo_2hop,
           mic_homo_1hop, mic_homo_2hop, x1, x2):
    return _run(hete_1hop, hete_2hop, drug_homo_1hop, drug_homo_2hop,
                mic_homo_1hop, mic_homo_2hop,
                x1.astype(jnp.int32), x2.astype(jnp.int32))
